# baseline (speedup 1.0000x reference)
import functools

import jax
import jax.numpy as jnp
from jax import lax
from jax.experimental import pallas as pl
from jax.experimental.pallas import tpu as pltpu
from jax.experimental.pallas import tpu_sc as plsc

N_NODES = 10000
D_FEAT = 128
HIDDEN = 64
N_CLASSES = 47
C_PAD = 48

NC, NS = 2, 16
NW = NC * NS
CHUNK = 128
NBUF = 8
ABUF = 4
N_PAD = 10240
RPT = N_PAD // NS


def _sc_mesh():
    return plsc.VectorSubcoreMesh(core_axis_name="c", subcore_axis_name="s")


def _split(per_s, f):
    nch0 = min(per_s, max(0, int(round(f * per_s / 8)) * 8))
    return nch0, per_s - nch0


def _make_deg_kernel(nch0, nch1):
    nch_max = max(nch0, nch1)

    @functools.partial(
        pl.kernel,
        out_type=jax.ShapeDtypeStruct((NC * N_PAD, 16), jnp.float32),
        mesh=_sc_mesh(),
        compiler_params=pltpu.CompilerParams(use_tc_tiling_on_sc=False),
        scratch_types=[
            pltpu.VMEM((nch_max, CHUNK), jnp.int32),
            pltpu.VMEM((CHUNK, 16), jnp.float32),
            pltpu.VMEM_SHARED((N_PAD, 16), jnp.float32),
            pltpu.SemaphoreType.DMA,
        ],
    )
    def k(ones_hbm, zeros_hbm, dst_hbm, out_hbm, dst_v, ones_v, acc, sem_s):
        c = lax.axis_index("c")
        s = lax.axis_index("s")
        rbase = s * RPT
        off = jnp.where(c == 0, s * nch0, NS * nch0 + s * nch1)
        ngroups = jnp.where(c == 0, nch0 // NBUF, nch1 // NBUF)
        pltpu.sync_copy(ones_hbm.at[pl.ds(0, CHUNK)], ones_v)
        pltpu.sync_copy(dst_hbm.at[pl.ds(off, nch_max)], dst_v)

        @pl.when(c == 0)
        def _():
            pltpu.sync_copy(ones_hbm.at[pl.ds(rbase, RPT)],
                            acc.at[pl.ds(rbase, RPT)])

        @pl.when(c != 0)
        def _():
            pltpu.sync_copy(zeros_hbm.at[pl.ds(rbase, RPT)],
                            acc.at[pl.ds(rbase, RPT)])

        plsc.subcore_barrier()

        def group(g, carry):
            base = g * NBUF
            scps = [pltpu.async_copy(ones_v, acc.at[dst_v.at[base + b]],
                                     sem_s, add=True)
                    for b in range(NBUF)]
            for cp in scps:
                cp.wait()
            return carry

        lax.fori_loop(0, ngroups, group, 0)
        plsc.subcore_barrier()
        pltpu.sync_copy(acc.at[pl.ds(rbase, RPT)],
                        out_hbm.at[pl.ds(c * N_PAD + rbase, RPT)])

    return k


def _make_agg_kernel(d, nch0, nch1):
    assert nch0 % (4 * ABUF) == 0 and nch1 % (4 * ABUF) == 0
    assert min(nch0, nch1) // ABUF >= 4

    @functools.partial(
        pl.kernel,
        out_type=jax.ShapeDtypeStruct((NC * N_PAD, d), jnp.float32),
        mesh=_sc_mesh(),
        compiler_params=pltpu.CompilerParams(use_tc_tiling_on_sc=False),
        scratch_types=[
            pltpu.VMEM((2, ABUF, CHUNK), jnp.int32),
            pltpu.VMEM((4, ABUF, CHUNK), jnp.int32),
            pltpu.VMEM((2, ABUF, CHUNK, d), jnp.float32),
            pltpu.VMEM_SHARED((N_PAD, d), jnp.float32),
            pltpu.SemaphoreType.DMA,
            pltpu.SemaphoreType.DMA,
            pltpu.SemaphoreType.DMA,
            pltpu.SemaphoreType.DMA,
        ],
    )
    def k(table_hbm, zeros_hbm, src_hbm, dst_hbm, out_hbm,
          srcb, dstb, rows, acc, sem_i0, sem_i1, sem_g, sem_s):
        c = lax.axis_index("c")
        s = lax.axis_index("s")
        rbase = s * RPT
        off = jnp.where(c == 0, s * nch0, NS * nch0 + s * nch1)
        ngroups = jnp.where(c == 0, nch0 // ABUF, nch1 // ABUF)

        def idx_start(j, sslot, dslot, sem):
            pltpu.async_copy(src_hbm.at[pl.ds(off + j * ABUF, ABUF)],
                             srcb.at[sslot], sem)
            pltpu.async_copy(dst_hbm.at[pl.ds(off + j * ABUF, ABUF)],
                             dstb.at[dslot], sem)

        def idx_wait(sem):
            for _ in range(2):
                pltpu.make_async_copy(src_hbm.at[pl.ds(0, ABUF)],
                                      srcb.at[0], sem).wait()

        def gather_start(p):
            for b in range(ABUF):
                pltpu.async_copy(table_hbm.at[srcb.at[p].at[b]],
                                 rows.at[p].at[b], sem_g)

        def gather_wait():
            for b in range(ABUF):
                pltpu.make_async_copy(table_hbm.at[pl.ds(0, CHUNK)],
                                      rows.at[0].at[b], sem_g).wait()

        def scatter_start(p, dslot):
            for b in range(ABUF):
                pltpu.async_copy(rows.at[p].at[b],
                                 acc.at[dstb.at[dslot].at[b]], sem_s, add=True)


        def scatter_wait():
            for b in range(ABUF):
                pltpu.make_async_copy(rows.at[0].at[b],
                                      acc.at[pl.ds(0, CHUNK)], sem_s).wait()

        @pl.when(c == 0)
        def _():
            pltpu.sync_copy(table_hbm.at[pl.ds(rbase, RPT)],
                            acc.at[pl.ds(rbase, RPT)])

        @pl.when(c != 0)
        def _():
            pltpu.sync_copy(zeros_hbm.at[pl.ds(rbase, RPT)],
                            acc.at[pl.ds(rbase, RPT)])

        plsc.subcore_barrier()

        idx_start(0, 0, 0, sem_i0)
        idx_start(1, 1, 1, sem_i1)
        idx_wait(sem_i0)
        gather_start(0)

        def quad(u, carry):
            for q in range(4):
                g = 4 * u + q
                p = q % 2
                sem_p = sem_i0 if p == 0 else sem_i1
                sem_o = sem_i1 if p == 0 else sem_i0
                gather_wait()
                if q == 0:
                    @pl.when(g > 0)
                    def _():
                        scatter_wait()
                else:
                    scatter_wait()

                @pl.when(g + 2 < ngroups)
                def _(sem_p=sem_p, g=g, p=p, q=q):
                    idx_start(g + 2, p, (q + 2) % 4, sem_p)

                scatter_start(p, q)

                @pl.when(g + 1 < ngroups)
                def _(sem_o=sem_o, p=p):
                    idx_wait(sem_o)
                    gather_start(1 - p)

            return carry

        lax.fori_loop(0, ngroups // 4, quad, 0)
        scatter_wait()
        plsc.subcore_barrier()
        pltpu.sync_copy(acc.at[pl.ds(rbase, RPT)],
                        out_hbm.at[pl.ds(c * N_PAD + rbase, RPT)])

    return k


_BM = 1024
_GRID = (N_PAD // _BM,)


def _dinv_of(dp_ref):
    dsum = dp_ref[0] + dp_ref[1]
    return lax.rsqrt(dsum[:, 0:1])


def _tc_h2_body(x_ref, w_ref, dp_ref, o_ref):
    dinv = _dinv_of(dp_ref)
    h = jnp.dot(x_ref[...], w_ref[...], preferred_element_type=jnp.float32)
    o_ref[...] = h * dinv


def _tc_mid_body(a_ref, dp_ref, b1_ref, w_ref, o_ref):
    dinv = _dinv_of(dp_ref)
    a = (a_ref[0] + a_ref[1]) * dinv + b1_ref[...]
    hr = jnp.maximum(a, 0.0)
    g = jnp.dot(hr, w_ref[...], preferred_element_type=jnp.float32)
    o_ref[...] = g * dinv


def _tc_out_body(a_ref, dp_ref, b2_ref, ls_ref, lg_ref):
    dinv = _dinv_of(dp_ref)
    logits = (a_ref[0] + a_ref[1]) * dinv + b2_ref[...]
    col = lax.broadcasted_iota(jnp.int32, (_BM, C_PAD), 1)
    valid = col < N_CLASSES
    m = jnp.max(jnp.where(valid, logits, -1e30), axis=1, keepdims=True)
    e = jnp.where(valid, jnp.exp(logits - m), 0.0)
    ssum = jnp.sum(e, axis=1, keepdims=True)
    ls_ref[...] = logits - m - jnp.log(ssum)
    lg_ref[...] = logits


def kernel(x, edge_index, W1, b1, W2, b2):
    src = edge_index[0].astype(jnp.int32)
    dst = edge_index[1].astype(jnp.int32)
    n_edges = src.shape[0]
    per_s = -(-n_edges // (NS * CHUNK))
    per_s = -(-per_s // 8) * 8
    e_pad = NS * per_s * CHUNK + per_s * CHUNK
    pad = jnp.full((e_pad - n_edges,), N_NODES, dtype=jnp.int32)
    src2 = jnp.concatenate([src, pad]).reshape((NS + 1) * per_s, CHUNK)
    dst2 = jnp.concatenate([dst, pad]).reshape((NS + 1) * per_s, CHUNK)

    xp = jnp.zeros((N_PAD, D_FEAT), jnp.float32).at[:N_NODES].set(x)
    ones16 = jnp.ones((N_PAD, 16), jnp.float32)
    zeros16 = jnp.zeros((N_PAD, 16), jnp.float32)
    zeros_h = jnp.zeros((N_PAD, HIDDEN), jnp.float32)
    zeros_c = jnp.zeros((N_PAD, C_PAD), jnp.float32)
    W2p = jnp.zeros((HIDDEN, C_PAD), jnp.float32).at[:, :N_CLASSES].set(W2)
    b1r = b1.reshape(1, HIDDEN)
    b2r = jnp.zeros((1, C_PAD), jnp.float32).at[0, :N_CLASSES].set(b2)

    d0, d1 = _split(per_s, 0.6)
    degp = _make_deg_kernel(d0, d1)(ones16, zeros16, dst2)
    degp = degp.reshape(NC, N_PAD, 16)

    h2 = pl.pallas_call(
        _tc_h2_body,
        grid=_GRID,
        in_specs=[
            pl.BlockSpec((_BM, D_FEAT), lambda i: (i, 0)),
            pl.BlockSpec((D_FEAT, HIDDEN), lambda i: (0, 0)),
            pl.BlockSpec((NC, _BM, 16), lambda i: (0, i, 0)),
        ],
        out_specs=pl.BlockSpec((_BM, HIDDEN), lambda i: (i, 0)),
        out_shape=jax.ShapeDtypeStruct((N_PAD, HIDDEN), jnp.float32),
    )(xp, W1, degp)

    a0, a1 = _split(per_s, 0.7)
    agg1 = _make_agg_kernel(HIDDEN, a0, a1)(h2, zeros_h, src2, dst2)
    agg1 = agg1.reshape(NC, N_PAD, HIDDEN)

    g2 = pl.pallas_call(
        _tc_mid_body,
        grid=_GRID,
        in_specs=[
            pl.BlockSpec((NC, _BM, HIDDEN), lambda i: (0, i, 0)),
            pl.BlockSpec((NC, _BM, 16), lambda i: (0, i, 0)),
            pl.BlockSpec((1, HIDDEN), lambda i: (0, 0)),
            pl.BlockSpec((HIDDEN, C_PAD), lambda i: (0, 0)),
        ],
        out_specs=pl.BlockSpec((_BM, C_PAD), lambda i: (i, 0)),
        out_shape=jax.ShapeDtypeStruct((N_PAD, C_PAD), jnp.float32),
    )(agg1, degp, b1r, W2p)

    b0, b1s = _split(per_s, 0.7)
    agg2 = _make_agg_kernel(C_PAD, b0, b1s)(g2, zeros_c, src2, dst2)
    agg2 = agg2.reshape(NC, N_PAD, C_PAD)

    ls, lg = pl.pallas_call(
        _tc_out_body,
        grid=_GRID,
        in_specs=[
            pl.BlockSpec((NC, _BM, C_PAD), lambda i: (0, i, 0)),
            pl.BlockSpec((NC, _BM, 16), lambda i: (0, i, 0)),
            pl.BlockSpec((1, C_PAD), lambda i: (0, 0)),
        ],
        out_specs=[
            pl.BlockSpec((_BM, C_PAD), lambda i: (i, 0)),
            pl.BlockSpec((_BM, C_PAD), lambda i: (i, 0)),
        ],
        out_shape=[
            jax.ShapeDtypeStruct((N_PAD, C_PAD), jnp.float32),
            jax.ShapeDtypeStruct((N_PAD, C_PAD), jnp.float32),
        ],
    )(agg2, degp, b2r)

    return (ls[:N_NODES, :N_CLASSES], lg[:N_NODES, :N_CLASSES])


# speedup vs baseline: 1.0682x; 1.0682x over previous
import functools

import jax
import jax.numpy as jnp
from jax import lax
from jax.experimental import pallas as pl
from jax.experimental.pallas import tpu as pltpu
from jax.experimental.pallas import tpu_sc as plsc

N_NODES = 10000
D_FEAT = 128
HIDDEN = 64
N_CLASSES = 47
C_PAD = 48

NC, NS = 2, 16
NW = NC * NS
CHUNK = 128
NBUF = 8
ABUF = 4
N_PAD = 10240
RPT = N_PAD // NS


def _sc_mesh():
    return plsc.VectorSubcoreMesh(core_axis_name="c", subcore_axis_name="s")


def _split(per_s, f):
    nch0 = min(per_s, max(0, int(round(f * per_s / 8)) * 8))
    return nch0, per_s - nch0


def _make_deg_kernel(per_s):
    nch = per_s // 2

    @functools.partial(
        pl.kernel,
        out_type=jax.ShapeDtypeStruct((NW, N_PAD), jnp.float32),
        mesh=_sc_mesh(),
        compiler_params=pltpu.CompilerParams(use_tc_tiling_on_sc=False,
                                             needs_layout_passes=False),
        scratch_types=[
            pltpu.VMEM((nch, CHUNK), jnp.int32),
            pltpu.VMEM((N_PAD,), jnp.float32),
        ],
    )
    def k(zeros_hbm, dst_hbm, out_hbm, dst_v, hist):
        c = lax.axis_index("c")
        s = lax.axis_index("s")
        wid = s * NC + c
        pltpu.sync_copy(dst_hbm.at[pl.ds(wid * nch, nch)], dst_v)
        pltpu.sync_copy(zeros_hbm, hist)
        ones = jnp.ones((16,), jnp.float32)

        def chunk(j, carry):
            for kk in range(CHUNK // 16):
                idx = dst_v[j, pl.ds(kk * 16, 16)]
                plsc.addupdate_scatter(hist, [idx], ones)
            return carry

        lax.fori_loop(0, nch, chunk, 0)
        pltpu.sync_copy(hist, out_hbm.at[wid])

    return k


def _make_agg_kernel(d, nch0, nch1, spmem_table=False):
    assert nch0 % (4 * ABUF) == 0 and nch1 % (4 * ABUF) == 0
    assert min(nch0, nch1) // ABUF >= 4

    @functools.partial(
        pl.kernel,
        out_type=jax.ShapeDtypeStruct((NC * N_PAD, d), jnp.float32),
        mesh=_sc_mesh(),
        compiler_params=pltpu.CompilerParams(use_tc_tiling_on_sc=False,
                                             needs_layout_passes=False),
        scratch_types=[
            pltpu.VMEM((2, ABUF, CHUNK), jnp.int32),
            pltpu.VMEM((4, ABUF, CHUNK), jnp.int32),
            pltpu.VMEM((2, ABUF, CHUNK, d), jnp.float32),
            pltpu.VMEM_SHARED((N_PAD, d), jnp.float32),
            pltpu.SemaphoreType.DMA,
            pltpu.SemaphoreType.DMA,
            pltpu.SemaphoreType.DMA,
            pltpu.SemaphoreType.DMA,
        ] + ([pltpu.VMEM_SHARED((N_PAD, d), jnp.float32)] if spmem_table else []),
    )
    def k(table_hbm, zeros_hbm, src_hbm, dst_hbm, out_hbm,
          srcb, dstb, rows, acc, sem_i0, sem_i1, sem_g, sem_s, *rest):
        table_s = rest[0] if spmem_table else None
        gsrc = table_s if spmem_table else table_hbm
        c = lax.axis_index("c")
        s = lax.axis_index("s")
        rbase = s * RPT
        off = jnp.where(c == 0, s * nch0, NS * nch0 + s * nch1)
        ngroups = jnp.where(c == 0, nch0 // ABUF, nch1 // ABUF)

        def idx_start(j, sslot, dslot, sem):
            pltpu.async_copy(src_hbm.at[pl.ds(off + j * ABUF, ABUF)],
                             srcb.at[sslot], sem)
            pltpu.async_copy(dst_hbm.at[pl.ds(off + j * ABUF, ABUF)],
                             dstb.at[dslot], sem)

        def idx_wait(sem):
            for _ in range(2):
                pltpu.make_async_copy(src_hbm.at[pl.ds(0, ABUF)],
                                      srcb.at[0], sem).wait()

        def gather_start(p):
            for b in range(ABUF):
                pltpu.async_copy(gsrc.at[srcb.at[p].at[b]],
                                 rows.at[p].at[b], sem_g)

        def gather_wait():
            for b in range(ABUF):
                pltpu.make_async_copy(table_hbm.at[pl.ds(0, CHUNK)],
                                      rows.at[0].at[b], sem_g).wait()

        def scatter_start(p, dslot):
            for b in range(ABUF):
                pltpu.async_copy(rows.at[p].at[b],
                                 acc.at[dstb.at[dslot].at[b]], sem_s, add=True)


        def scatter_wait():
            for b in range(ABUF):
                pltpu.make_async_copy(rows.at[0].at[b],
                                      acc.at[pl.ds(0, CHUNK)], sem_s).wait()

        if spmem_table:
            pltpu.sync_copy(table_hbm.at[pl.ds(rbase, RPT)],
                            table_s.at[pl.ds(rbase, RPT)])

        @pl.when(c == 0)
        def _():
            pltpu.sync_copy(table_hbm.at[pl.ds(rbase, RPT)],
                            acc.at[pl.ds(rbase, RPT)])

        @pl.when(c != 0)
        def _():
            pltpu.sync_copy(zeros_hbm.at[pl.ds(rbase, RPT)],
                            acc.at[pl.ds(rbase, RPT)])

        plsc.subcore_barrier()

        idx_start(0, 0, 0, sem_i0)
        idx_start(1, 1, 1, sem_i1)
        idx_wait(sem_i0)
        gather_start(0)

        def quad(u, carry):
            for q in range(4):
                g = 4 * u + q
                p = q % 2
                sem_p = sem_i0 if p == 0 else sem_i1
                sem_o = sem_i1 if p == 0 else sem_i0
                gather_wait()
                if q == 0:
                    @pl.when(g > 0)
                    def _():
                        scatter_wait()
                else:
                    scatter_wait()

                @pl.when(g + 2 < ngroups)
                def _(sem_p=sem_p, g=g, p=p, q=q):
                    idx_start(g + 2, p, (q + 2) % 4, sem_p)

                scatter_start(p, q)

                @pl.when(g + 1 < ngroups)
                def _(sem_o=sem_o, p=p):
                    idx_wait(sem_o)
                    gather_start(1 - p)

            return carry

        lax.fori_loop(0, ngroups // 4, quad, 0)
        scatter_wait()
        plsc.subcore_barrier()
        pltpu.sync_copy(acc.at[pl.ds(rbase, RPT)],
                        out_hbm.at[pl.ds(c * N_PAD + rbase, RPT)])

    return k


_BM = 1024
_GRID = (N_PAD // _BM,)


def _tc_h2_body(x_ref, w_ref, dp_ref, o_ref, dv_ref):
    deg = jnp.sum(dp_ref[...], axis=0, keepdims=True) + 1.0
    dinv = jnp.transpose(lax.rsqrt(deg))
    h = jnp.dot(x_ref[...], w_ref[...], preferred_element_type=jnp.float32)
    o_ref[...] = h * dinv
    dv_ref[...] = jnp.broadcast_to(dinv, (dinv.shape[0], 16))


def _tc_mid_body(a_ref, dv_ref, b1_ref, w_ref, o_ref):
    dinv = dv_ref[:, 0:1]
    a = (a_ref[0] + a_ref[1]) * dinv + b1_ref[...]
    hr = jnp.maximum(a, 0.0)
    g = jnp.dot(hr, w_ref[...], preferred_element_type=jnp.float32)
    o_ref[...] = g * dinv


def _tc_out_body(a_ref, dv_ref, b2_ref, ls_ref, lg_ref):
    dinv = dv_ref[:, 0:1]
    logits = (a_ref[0] + a_ref[1]) * dinv + b2_ref[...]
    col = lax.broadcasted_iota(jnp.int32, (_BM, C_PAD), 1)
    valid = col < N_CLASSES
    m = jnp.max(jnp.where(valid, logits, -1e30), axis=1, keepdims=True)
    e = jnp.where(valid, jnp.exp(logits - m), 0.0)
    ssum = jnp.sum(e, axis=1, keepdims=True)
    ls_ref[...] = logits - m - jnp.log(ssum)
    lg_ref[...] = logits


def kernel(x, edge_index, W1, b1, W2, b2):
    src = edge_index[0].astype(jnp.int32)
    dst = edge_index[1].astype(jnp.int32)
    n_edges = src.shape[0]
    per_s = -(-n_edges // (NS * CHUNK))
    per_s = -(-per_s // 8) * 8
    e_pad = NS * per_s * CHUNK + per_s * CHUNK
    pad = jnp.full((e_pad - n_edges,), N_NODES, dtype=jnp.int32)
    src2 = jnp.concatenate([src, pad]).reshape((NS + 1) * per_s, CHUNK)
    dst2 = jnp.concatenate([dst, pad]).reshape((NS + 1) * per_s, CHUNK)

    xp = jnp.zeros((N_PAD, D_FEAT), jnp.float32).at[:N_NODES].set(x)
    zeros1 = jnp.zeros((N_PAD,), jnp.float32)
    zeros_h = jnp.zeros((N_PAD, HIDDEN), jnp.float32)
    zeros_c = jnp.zeros((N_PAD, C_PAD), jnp.float32)
    W2p = jnp.zeros((HIDDEN, C_PAD), jnp.float32).at[:, :N_CLASSES].set(W2)
    b1r = b1.reshape(1, HIDDEN)
    b2r = jnp.zeros((1, C_PAD), jnp.float32).at[0, :N_CLASSES].set(b2)

    degp = _make_deg_kernel(per_s)(zeros1, dst2)

    h2, dinv16 = pl.pallas_call(
        _tc_h2_body,
        grid=_GRID,
        in_specs=[
            pl.BlockSpec((_BM, D_FEAT), lambda i: (i, 0)),
            pl.BlockSpec((D_FEAT, HIDDEN), lambda i: (0, 0)),
            pl.BlockSpec((NW, _BM), lambda i: (0, i)),
        ],
        out_specs=[
            pl.BlockSpec((_BM, HIDDEN), lambda i: (i, 0)),
            pl.BlockSpec((_BM, 16), lambda i: (i, 0)),
        ],
        out_shape=[
            jax.ShapeDtypeStruct((N_PAD, HIDDEN), jnp.float32),
            jax.ShapeDtypeStruct((N_PAD, 16), jnp.float32),
        ],
    )(xp, W1, degp)

    a0, a1 = _split(per_s, 0.9)
    agg1 = _make_agg_kernel(HIDDEN, a0, a1)(h2, zeros_h, src2, dst2)
    agg1 = agg1.reshape(NC, N_PAD, HIDDEN)

    g2 = pl.pallas_call(
        _tc_mid_body,
        grid=_GRID,
        in_specs=[
            pl.BlockSpec((NC, _BM, HIDDEN), lambda i: (0, i, 0)),
            pl.BlockSpec((_BM, 16), lambda i: (i, 0)),
            pl.BlockSpec((1, HIDDEN), lambda i: (0, 0)),
            pl.BlockSpec((HIDDEN, C_PAD), lambda i: (0, 0)),
        ],
        out_specs=pl.BlockSpec((_BM, C_PAD), lambda i: (i, 0)),
        out_shape=jax.ShapeDtypeStruct((N_PAD, C_PAD), jnp.float32),
    )(agg1, dinv16, b1r, W2p)

    b0, b1s = _split(per_s, 0.9)
    agg2 = _make_agg_kernel(C_PAD, b0, b1s)(g2, zeros_c, src2, dst2)
    agg2 = agg2.reshape(NC, N_PAD, C_PAD)

    ls, lg = pl.pallas_call(
        _tc_out_body,
        grid=_GRID,
        in_specs=[
            pl.BlockSpec((NC, _BM, C_PAD), lambda i: (0, i, 0)),
            pl.BlockSpec((_BM, 16), lambda i: (i, 0)),
            pl.BlockSpec((1, C_PAD), lambda i: (0, 0)),
        ],
        out_specs=[
            pl.BlockSpec((_BM, C_PAD), lambda i: (i, 0)),
            pl.BlockSpec((_BM, C_PAD), lambda i: (i, 0)),
        ],
        out_shape=[
            jax.ShapeDtypeStruct((N_PAD, C_PAD), jnp.float32),
            jax.ShapeDtypeStruct((N_PAD, C_PAD), jnp.float32),
        ],
    )(agg2, dinv16, b2r)

    return (ls[:N_NODES, :N_CLASSES], lg[:N_NODES, :N_CLASSES])
